# Initial kernel scaffold; baseline (speedup 1.0000x reference)
import jax
import jax.numpy as jnp
from jax.experimental import pallas as pl


def kernel(input, adj, W, a):
    raise NotImplementedError("write your pallas kernel here")


# speedup vs baseline: 10.9061x; 10.9061x over previous
import functools

import jax
import jax.numpy as jnp
from jax.experimental import pallas as pl
from jax.experimental.pallas import tpu as pltpu

PARTS = 3
ALPHA = 0.2
F = 128
V = 25
NEG = -1e30


def _gat_kernel(x_ref, wt_ref, a6_ref, adj_ref, out_ref, *, tb):
    wt = wt_ref[...]
    a6 = a6_ref[...]
    masks = [adj_ref[k] != 0 for k in range(PARTS)]
    for t in range(tb):
        xb = x_ref[0, :, t, :]
        h = jnp.dot(wt, xb, preferred_element_type=jnp.float32)
        s_cols = jax.lax.dot_general(h, a6, (((0,), (0,)), ((), ())),
                                     preferred_element_type=jnp.float32)
        s_rows = jax.lax.dot_general(a6, h, (((0,), (0,)), ((), ())),
                                     preferred_element_type=jnp.float32)
        es = []
        for k in range(PARTS):
            e_k = s_cols[:, k:k + 1] + s_rows[3 + k:4 + k, :]
            e_k = jnp.where(e_k >= 0, e_k, ALPHA * e_k)
            es.append(jnp.where(masks[k], e_k, NEG))
        m = jnp.max(jnp.maximum(jnp.maximum(es[0], es[1]), es[2]),
                    axis=0, keepdims=True)
        exs = [jnp.where(masks[k], jnp.exp(es[k] - m), 0.0) for k in range(PARTS)]
        den = (jnp.sum(exs[0], axis=0, keepdims=True)
               + jnp.sum(exs[1], axis=0, keepdims=True)
               + jnp.sum(exs[2], axis=0, keepdims=True))
        inv = 1.0 / jnp.maximum(den, 1e-30)
        agg = (jnp.dot(h, exs[0] * inv, preferred_element_type=jnp.float32)
               + jnp.dot(h, exs[1] * inv, preferred_element_type=jnp.float32)
               + jnp.dot(h, exs[2] * inv, preferred_element_type=jnp.float32))
        out_ref[0, :, t, :] = jnp.where(agg > 0, agg, jnp.exp(agg) - 1.0)


@jax.jit
def kernel(input, adj, W, a):
    N, C, T, Vv = input.shape
    tb = 8
    wt = W.T
    a6 = jnp.concatenate([a[:, :F, 0].T, a[:, F:, 0].T], axis=1)
    grid = (N, T // tb)
    out = pl.pallas_call(
        functools.partial(_gat_kernel, tb=tb),
        grid=grid,
        in_specs=[
            pl.BlockSpec((1, C, tb, Vv), lambda n, j: (n, 0, j, 0)),
            pl.BlockSpec((F, C), lambda n, j: (0, 0)),
            pl.BlockSpec((C, 2 * PARTS), lambda n, j: (0, 0)),
            pl.BlockSpec((PARTS, Vv, Vv), lambda n, j: (0, 0, 0)),
        ],
        out_specs=pl.BlockSpec((1, F, tb, Vv), lambda n, j: (n, 0, j, 0)),
        out_shape=jax.ShapeDtypeStruct((N, F, T, Vv), jnp.float32),
        compiler_params=pltpu.CompilerParams(
            dimension_semantics=("parallel", "parallel")),
    )(input, wt, a6, adj)
    return out


# speedup vs baseline: 37.2714x; 3.4175x over previous
import functools

import jax
import jax.numpy as jnp
from jax.experimental import pallas as pl
from jax.experimental.pallas import tpu as pltpu

PARTS = 3
ALPHA = 0.2
F = 128
V = 25
VP = 32
NEG = -1e30


def _gat_kernel(x_ref, w_ref, a6_ref, adj_ref, out_ref, *, tb):
    x2 = x_ref[0]
    w = w_ref[...]
    a6 = a6_ref[...]
    masks = [adj_ref[k] != 0 for k in range(PARTS)]
    h = jnp.dot(x2, w, preferred_element_type=jnp.float32)
    s_cols = jnp.dot(h, a6, preferred_element_type=jnp.float32)
    s_rows = jax.lax.dot_general(a6, h, (((0,), (1,)), ((), ())),
                                 preferred_element_type=jnp.float32)
    for t in range(tb):
        lo = t * VP
        hb = h[lo:lo + VP, :]
        sc = s_cols[lo:lo + VP, :]
        es = []
        for k in range(PARTS):
            e_k = sc[:, k:k + 1] + s_rows[3 + k:4 + k, lo:lo + VP]
            e_k = jnp.where(e_k >= 0, e_k, ALPHA * e_k)
            es.append(jnp.where(masks[k], e_k, NEG))
        m = jnp.max(jnp.maximum(jnp.maximum(es[0], es[1]), es[2]),
                    axis=0, keepdims=True)
        exs = [jnp.where(masks[k], jnp.exp(es[k] - m), 0.0) for k in range(PARTS)]
        den = (jnp.sum(exs[0], axis=0, keepdims=True)
               + jnp.sum(exs[1], axis=0, keepdims=True)
               + jnp.sum(exs[2], axis=0, keepdims=True))
        inv = 1.0 / jnp.maximum(den, 1e-30)
        agg = (jax.lax.dot_general(exs[0] * inv, hb, (((0,), (0,)), ((), ())),
                                   preferred_element_type=jnp.float32)
               + jax.lax.dot_general(exs[1] * inv, hb, (((0,), (0,)), ((), ())),
                                     preferred_element_type=jnp.float32)
               + jax.lax.dot_general(exs[2] * inv, hb, (((0,), (0,)), ((), ())),
                                     preferred_element_type=jnp.float32))
        out_ref[0, lo:lo + VP, :] = jnp.where(agg > 0, agg, jnp.exp(agg) - 1.0)


@jax.jit
def kernel(input, adj, W, a):
    N, C, T, Vv = input.shape
    tb = 16
    xp = jnp.pad(input, ((0, 0), (0, 0), (0, 0), (0, VP - Vv)))
    xr = xp.transpose(0, 2, 3, 1).reshape(N, T * VP, C)
    adjp = jnp.pad(adj, ((0, 0), (0, VP - Vv), (0, VP - Vv)))
    a6 = jnp.concatenate(
        [a[:, :F, 0].T, a[:, F:, 0].T, jnp.zeros((C, 2), jnp.float32)],
        axis=1)
    grid = (N, T // tb)
    out = pl.pallas_call(
        functools.partial(_gat_kernel, tb=tb),
        grid=grid,
        in_specs=[
            pl.BlockSpec((1, tb * VP, C), lambda n, j: (n, j, 0)),
            pl.BlockSpec((C, F), lambda n, j: (0, 0)),
            pl.BlockSpec((C, 8), lambda n, j: (0, 0)),
            pl.BlockSpec((PARTS, VP, VP), lambda n, j: (0, 0, 0)),
        ],
        out_specs=pl.BlockSpec((1, tb * VP, F), lambda n, j: (n, j, 0)),
        out_shape=jax.ShapeDtypeStruct((N, T * VP, F), jnp.float32),
        compiler_params=pltpu.CompilerParams(
            dimension_semantics=("parallel", "parallel")),
    )(xr, W, a6, adjp)
    out = out.reshape(N, T, VP, F)[:, :, :Vv, :].transpose(0, 3, 1, 2)
    return out
